# baseline (speedup 1.0000x reference)
import functools

import jax
import jax.numpy as jnp
from jax import lax
from jax.experimental import pallas as pl
from jax.experimental.pallas import tpu as pltpu
from jax.experimental.pallas import tpu_sc as plsc

N = 10000
D = 128
E = 320000
NC = 2
NS = 16
NW = NC * NS
EPW = E // NW
K = 128
NCHUNK = 80
EPWP = NCHUNK * K
EPAD = NW * EPWP
IH = 40
NP = N + 8
KH = 2000
NCHUNK_H = EPW // KH
ROWS_PT = 624

_HIGH = lax.Precision.HIGHEST


def _sc_mesh():
    return plsc.VectorSubcoreMesh(core_axis_name="c", subcore_axis_name="s")


def _sc_degrees(src, dst):
    @functools.partial(
        pl.kernel,
        out_type=jax.ShapeDtypeStruct((NC, 2, N, 16), jnp.float32),
        mesh=_sc_mesh(),
        compiler_params=pltpu.CompilerParams(use_tc_tiling_on_sc=False),
        scratch_types=[
            pltpu.VMEM((KH,), jnp.int32),
            pltpu.VMEM((KH, 16), jnp.float32),
            pltpu.VMEM_SHARED((N, 16), jnp.float32),
            pltpu.VMEM_SHARED((N, 16), jnp.float32),
        ],
    )
    def hist(src_hbm, dst_hbm, out_hbm, idx_v, ones_v, acc_s, acc_d):
        cid = lax.axis_index("c")
        sid = lax.axis_index("s")
        wid = sid * NC + cid
        one = jnp.ones((16,), jnp.float32)
        zero = jnp.zeros((16,), jnp.float32)

        @pl.loop(0, KH)
        def _(r):
            ones_v.at[r][...] = one

        row0 = sid * ROWS_PT

        @pl.loop(0, 8)
        def _(r):
            ones_v.at[r][...] = zero

        @pl.loop(0, 640, step=8)
        def _(r):
            pltpu.sync_copy(ones_v.at[pl.ds(0, 8)], acc_s.at[pl.ds(row0 + r, 8)])
            pltpu.sync_copy(ones_v.at[pl.ds(0, 8)], acc_d.at[pl.ds(row0 + r, 8)])

        @pl.loop(0, 8)
        def _(r):
            ones_v.at[r][...] = one

        plsc.subcore_barrier()

        base = wid * EPW

        @pl.loop(0, NCHUNK_H)
        def _(chunk):
            off = base + chunk * KH
            pltpu.sync_copy(src_hbm.at[pl.ds(off, KH)], idx_v)
            pltpu.sync_copy(ones_v, acc_s.at[idx_v], add=True)
            pltpu.sync_copy(dst_hbm.at[pl.ds(off, KH)], idx_v)
            pltpu.sync_copy(ones_v, acc_d.at[idx_v], add=True)

        plsc.subcore_barrier()
        pltpu.sync_copy(acc_s.at[pl.ds(row0, ROWS_PT)],
                        out_hbm.at[cid, 0, pl.ds(row0, ROWS_PT)])
        pltpu.sync_copy(acc_d.at[pl.ds(row0, ROWS_PT)],
                        out_hbm.at[cid, 1, pl.ds(row0, ROWS_PT)])

        @pl.when(sid == NS - 1)
        def _():
            pltpu.sync_copy(acc_s.at[pl.ds(NS * ROWS_PT, N - NS * ROWS_PT)],
                            out_hbm.at[cid, 0, pl.ds(NS * ROWS_PT, N - NS * ROWS_PT)])
            pltpu.sync_copy(acc_d.at[pl.ds(NS * ROWS_PT, N - NS * ROWS_PT)],
                            out_hbm.at[cid, 1, pl.ds(NS * ROWS_PT, N - NS * ROWS_PT)])

    return hist(src, dst)


def _sc_propagate(x, src3, dst3):
    @functools.partial(
        pl.kernel,
        out_type=jax.ShapeDtypeStruct((NC, N, D), jnp.float32),
        mesh=_sc_mesh(),
        scratch_types=[
            pltpu.VMEM((IH, K), jnp.int32),
            pltpu.VMEM((IH, K), jnp.int32),
            pltpu.VMEM((K, D), jnp.float32),
            pltpu.VMEM((K, D), jnp.float32),
            pltpu.VMEM_SHARED((NP, D), jnp.float32),
            pltpu.SemaphoreType.DMA,
            pltpu.SemaphoreType.DMA,
        ],
    )
    def prop(x_hbm, src_hbm, dst_hbm, out_hbm, src_v, dst_v, rows0, rows1,
             acc_sh, sem0, sem1):
        cid = lax.axis_index("c")
        sid = lax.axis_index("s")
        wid = sid * NC + cid
        zero = jnp.zeros((16,), jnp.float32)

        @pl.loop(0, 8)
        def _(r):
            @pl.loop(0, D, step=16)
            def _(c0):
                rows0.at[r, pl.ds(c0, 16)][...] = zero

        row0 = sid * ROWS_PT

        @pl.loop(0, 640, step=8)
        def _(r):
            pltpu.sync_copy(rows0.at[pl.ds(0, 8)], acc_sh.at[pl.ds(row0 + r, 8)])

        plsc.subcore_barrier()

        for phase in range(2):
            pltpu.sync_copy(src_hbm.at[wid, pl.ds(phase * IH, IH)], src_v)
            pltpu.sync_copy(dst_hbm.at[wid, pl.ds(phase * IH, IH)], dst_v)
            pltpu.async_copy(x_hbm.at[src_v.at[0]], rows0, sem0)

            @pl.loop(0, IH, step=2)
            def _(j):
                pltpu.async_copy(x_hbm.at[src_v.at[j + 1]], rows1, sem1)
                pltpu.make_async_copy(x_hbm.at[src_v.at[j]], rows0, sem0).wait()
                pltpu.sync_copy(rows0, acc_sh.at[dst_v.at[j]], add=True)

                @pl.when(j < IH - 2)
                def _():
                    pltpu.async_copy(x_hbm.at[src_v.at[j + 2]], rows0, sem0)

                pltpu.make_async_copy(x_hbm.at[src_v.at[j + 1]], rows1,
                                      sem1).wait()
                pltpu.sync_copy(rows1, acc_sh.at[dst_v.at[j + 1]], add=True)

        plsc.subcore_barrier()
        pltpu.sync_copy(acc_sh.at[pl.ds(row0, ROWS_PT)],
                        out_hbm.at[cid, pl.ds(row0, ROWS_PT)])

        @pl.when(sid == NS - 1)
        def _():
            pltpu.sync_copy(acc_sh.at[pl.ds(NS * ROWS_PT, N - NS * ROWS_PT)],
                            out_hbm.at[cid, pl.ds(NS * ROWS_PT, N - NS * ROWS_PT)])

    return prop(x, src3, dst3)


def _tc_prepare(counts, features):
    def body(counts_ref, feat_ref, h_ref, no_ref, ni_ref):
        c = counts_ref[...]
        deg_o = c[0, 0, :, 0:1] + c[1, 0, :, 0:1]
        deg_i = c[0, 1, :, 0:1] + c[1, 1, :, 0:1]
        no = lax.rsqrt(jnp.maximum(deg_o, 1.0))
        ni = lax.rsqrt(jnp.maximum(deg_i, 1.0))
        h_ref[...] = feat_ref[...] * no
        no_ref[...] = no
        ni_ref[...] = ni

    return pl.pallas_call(
        body,
        out_shape=(
            jax.ShapeDtypeStruct((NP, D), jnp.float32),
            jax.ShapeDtypeStruct((N, 1), jnp.float32),
            jax.ShapeDtypeStruct((N, 1), jnp.float32),
        ),
        grid=(N // _RB,),
        in_specs=[
            pl.BlockSpec((NC, 2, _RB, 16), lambda i: (0, 0, i, 0)),
            pl.BlockSpec((_RB, D), lambda i: (i, 0)),
        ],
        out_specs=(
            pl.BlockSpec((_RB, D), lambda i: (i, 0)),
            pl.BlockSpec((_RB, 1), lambda i: (i, 0)),
            pl.BlockSpec((_RB, 1), lambda i: (i, 0)),
        ),
    )(counts, features)


_RB = 2000


def _w_spec(shape):
    return pl.BlockSpec(shape, lambda i: (0,) * len(shape))


def _tc_mid(p, norm_in, norm_out, W10, b10, W11, b11, W20, W21):
    def body(p_ref, ni_ref, no_ref, W10_r, b10_r, W11_r, b11_r, W20_r, W21_r,
             q_ref):
        agg = (p_ref[0] + p_ref[1]) * ni_ref[...]
        o0 = jnp.maximum(
            jnp.dot(agg, W10_r[...], preferred_element_type=jnp.float32,
                    precision=_HIGH) + b10_r[...], 0.0)
        o1 = jnp.maximum(
            jnp.dot(agg, W11_r[...], preferred_element_type=jnp.float32,
                    precision=_HIGH) + b11_r[...], 0.0)
        q0 = jnp.dot(o0, W20_r[...], preferred_element_type=jnp.float32,
                     precision=_HIGH)
        q1 = jnp.dot(o1, W21_r[...], preferred_element_type=jnp.float32,
                     precision=_HIGH)
        q_ref[...] = jnp.concatenate([q0, q1], axis=1) * no_ref[...]

    return pl.pallas_call(
        body,
        grid=(N // _RB,),
        in_specs=[
            pl.BlockSpec((NC, _RB, D), lambda i: (0, i, 0)),
            pl.BlockSpec((_RB, 1), lambda i: (i, 0)),
            pl.BlockSpec((_RB, 1), lambda i: (i, 0)),
            _w_spec((D, D)), _w_spec((1, D)),
            _w_spec((D, D)), _w_spec((1, D)),
            _w_spec((D, D // 2)), _w_spec((D, D // 2)),
        ],
        out_specs=pl.BlockSpec((_RB, D), lambda i: (i, 0)),
        out_shape=jax.ShapeDtypeStruct((NP, D), jnp.float32),
    )(p, norm_in, norm_out, W10, b10, W11, b11, W20, W21)


def _tc_final(p2, norm_in, features, b2cat, ln_g, ln_b, Wf1, bf1, Wf2, bf2):
    def body(p_ref, ni_ref, feat_ref, b2_r, g_r, b_r, Wf1_r, bf1_r, Wf2_r,
             bf2_r, out_ref):
        hcat = (p_ref[0] + p_ref[1]) * ni_ref[...] + b2_r[...]
        mu = jnp.mean(hcat, axis=1, keepdims=True)
        xc = hcat - mu
        var = jnp.mean(xc * xc, axis=1, keepdims=True)
        ln = xc * lax.rsqrt(var + 1e-5) * g_r[...] + b_r[...]
        h2 = feat_ref[...] + ln
        ff = jnp.maximum(
            jnp.dot(h2, Wf1_r[...], preferred_element_type=jnp.float32,
                    precision=_HIGH) + bf1_r[...], 0.0)
        ff = jnp.dot(ff, Wf2_r[...], preferred_element_type=jnp.float32,
                     precision=_HIGH) + bf2_r[...]
        out_ref[...] = h2 + ff

    return pl.pallas_call(
        body,
        grid=(N // _RB,),
        in_specs=[
            pl.BlockSpec((NC, _RB, D), lambda i: (0, i, 0)),
            pl.BlockSpec((_RB, 1), lambda i: (i, 0)),
            pl.BlockSpec((_RB, D), lambda i: (i, 0)),
            _w_spec((1, D)), _w_spec((1, D)), _w_spec((1, D)),
            _w_spec((D, D)), _w_spec((1, D)),
            _w_spec((D, D)), _w_spec((1, D)),
        ],
        out_specs=pl.BlockSpec((_RB, D), lambda i: (i, 0)),
        out_shape=jax.ShapeDtypeStruct((N, D), jnp.float32),
    )(p2, norm_in, features, b2cat, ln_g, ln_b, Wf1, bf1, Wf2, bf2)


def kernel(features, edge_index, W1_0, b1_0, W2_0, b2_0, W1_1, b1_1, W2_1,
           b2_1, ln_g, ln_b, Wf1, bf1, Wf2, bf2):
    src = edge_index[0]
    dst = edge_index[1]

    pad = jnp.full((EPAD - E,), N, jnp.int32)
    src3 = jnp.concatenate([src, pad]).reshape(NW, NCHUNK, K)
    dst3 = jnp.concatenate([dst, pad]).reshape(NW, NCHUNK, K)

    counts = _sc_degrees(src, dst)
    h_scaled, norm_out, norm_in = _tc_prepare(counts, features)

    p1 = _sc_propagate(h_scaled, src3, dst3)
    q = _tc_mid(p1, norm_in, norm_out,
                W1_0, b1_0.reshape(1, D), W1_1, b1_1.reshape(1, D), W2_0, W2_1)

    p2 = _sc_propagate(q, src3, dst3)
    b2cat = jnp.concatenate([b2_0, b2_1]).reshape(1, D)
    out = _tc_final(p2, norm_in, features, b2cat, ln_g.reshape(1, D),
                    ln_b.reshape(1, D), Wf1, bf1.reshape(1, D), Wf2,
                    bf2.reshape(1, D))
    return out


# speedup vs baseline: 3.0538x; 3.0538x over previous
import functools

import jax
import jax.numpy as jnp
from jax import lax
from jax.experimental import pallas as pl
from jax.experimental.pallas import tpu as pltpu
from jax.experimental.pallas import tpu_sc as plsc

N = 10000
D = 128
E = 320000
NC = 2
NS = 16
NW = NC * NS
EPW = E // NW
K = 128
NCHUNK = 80
EPWP = NCHUNK * K
EPAD = NW * EPWP
IH = 40
PADR = 512
NP = N + PADR
KH = 2000
NCHUNK_H = EPW // KH
ROWS_PT = 624

_HIGH = lax.Precision.HIGHEST


def _sc_mesh():
    return plsc.VectorSubcoreMesh(core_axis_name="c", subcore_axis_name="s")


def _sc_degrees(src, dst):
    @functools.partial(
        pl.kernel,
        out_type=jax.ShapeDtypeStruct((NC, 2, N, 16), jnp.float32),
        mesh=_sc_mesh(),
        compiler_params=pltpu.CompilerParams(use_tc_tiling_on_sc=False),
        scratch_types=[
            pltpu.VMEM((KH,), jnp.int32),
            pltpu.VMEM((KH, 16), jnp.float32),
            pltpu.VMEM_SHARED((N, 16), jnp.float32),
            pltpu.VMEM_SHARED((N, 16), jnp.float32),
        ],
    )
    def hist(src_hbm, dst_hbm, out_hbm, idx_v, ones_v, acc_s, acc_d):
        cid = lax.axis_index("c")
        sid = lax.axis_index("s")
        wid = sid * NC + cid
        one = jnp.ones((16,), jnp.float32)
        zero = jnp.zeros((16,), jnp.float32)

        @pl.loop(0, KH)
        def _(r):
            ones_v.at[r][...] = one

        row0 = sid * ROWS_PT

        @pl.loop(0, 8)
        def _(r):
            ones_v.at[r][...] = zero

        @pl.loop(0, 640, step=8)
        def _(r):
            pltpu.sync_copy(ones_v.at[pl.ds(0, 8)], acc_s.at[pl.ds(row0 + r, 8)])
            pltpu.sync_copy(ones_v.at[pl.ds(0, 8)], acc_d.at[pl.ds(row0 + r, 8)])

        @pl.loop(0, 8)
        def _(r):
            ones_v.at[r][...] = one

        plsc.subcore_barrier()

        base = wid * EPW

        @pl.loop(0, NCHUNK_H)
        def _(chunk):
            off = base + chunk * KH
            pltpu.sync_copy(src_hbm.at[pl.ds(off, KH)], idx_v)
            pltpu.sync_copy(ones_v, acc_s.at[idx_v], add=True)
            pltpu.sync_copy(dst_hbm.at[pl.ds(off, KH)], idx_v)
            pltpu.sync_copy(ones_v, acc_d.at[idx_v], add=True)

        plsc.subcore_barrier()
        pltpu.sync_copy(acc_s.at[pl.ds(row0, ROWS_PT)],
                        out_hbm.at[cid, 0, pl.ds(row0, ROWS_PT)])
        pltpu.sync_copy(acc_d.at[pl.ds(row0, ROWS_PT)],
                        out_hbm.at[cid, 1, pl.ds(row0, ROWS_PT)])

        @pl.when(sid == NS - 1)
        def _():
            pltpu.sync_copy(acc_s.at[pl.ds(NS * ROWS_PT, N - NS * ROWS_PT)],
                            out_hbm.at[cid, 0, pl.ds(NS * ROWS_PT, N - NS * ROWS_PT)])
            pltpu.sync_copy(acc_d.at[pl.ds(NS * ROWS_PT, N - NS * ROWS_PT)],
                            out_hbm.at[cid, 1, pl.ds(NS * ROWS_PT, N - NS * ROWS_PT)])

    return hist(src, dst)


def _sc_propagate(x, src3, dst3):
    @functools.partial(
        pl.kernel,
        out_type=jax.ShapeDtypeStruct((NC, N, D), jnp.float32),
        mesh=_sc_mesh(),
        scratch_types=[
            pltpu.VMEM((IH, K), jnp.int32),
            pltpu.VMEM((IH, K), jnp.int32),
            pltpu.VMEM((K, D), jnp.float32),
            pltpu.VMEM((K, D), jnp.float32),
            pltpu.VMEM_SHARED((NP, D), jnp.float32),
            pltpu.SemaphoreType.DMA,
            pltpu.SemaphoreType.DMA,
        ],
    )
    def prop(x_hbm, src_hbm, dst_hbm, out_hbm, src_v, dst_v, rows0, rows1,
             acc_sh, sem0, sem1):
        cid = lax.axis_index("c")
        sid = lax.axis_index("s")
        wid = sid * NC + cid
        zero = jnp.zeros((16,), jnp.float32)

        @pl.loop(0, 8)
        def _(r):
            @pl.loop(0, D, step=16)
            def _(c0):
                rows0.at[r, pl.ds(c0, 16)][...] = zero

        row0 = sid * ROWS_PT

        @pl.loop(0, 640, step=8)
        def _(r):
            pltpu.sync_copy(rows0.at[pl.ds(0, 8)], acc_sh.at[pl.ds(row0 + r, 8)])

        plsc.subcore_barrier()

        for phase in range(2):
            pltpu.sync_copy(src_hbm.at[wid, pl.ds(phase * IH, IH)], src_v)
            pltpu.sync_copy(dst_hbm.at[wid, pl.ds(phase * IH, IH)], dst_v)
            pltpu.async_copy(x_hbm.at[src_v.at[0]], rows0, sem0)

            @pl.loop(0, IH, step=2)
            def _(j):
                pltpu.async_copy(x_hbm.at[src_v.at[j + 1]], rows1, sem1)
                pltpu.make_async_copy(x_hbm.at[src_v.at[j]], rows0, sem0).wait()
                pltpu.sync_copy(rows0, acc_sh.at[dst_v.at[j]], add=True)

                @pl.when(j < IH - 2)
                def _():
                    pltpu.async_copy(x_hbm.at[src_v.at[j + 2]], rows0, sem0)

                pltpu.make_async_copy(x_hbm.at[src_v.at[j + 1]], rows1,
                                      sem1).wait()
                pltpu.sync_copy(rows1, acc_sh.at[dst_v.at[j + 1]], add=True)

        plsc.subcore_barrier()
        pltpu.sync_copy(acc_sh.at[pl.ds(row0, ROWS_PT)],
                        out_hbm.at[cid, pl.ds(row0, ROWS_PT)])

        @pl.when(sid == NS - 1)
        def _():
            pltpu.sync_copy(acc_sh.at[pl.ds(NS * ROWS_PT, N - NS * ROWS_PT)],
                            out_hbm.at[cid, pl.ds(NS * ROWS_PT, N - NS * ROWS_PT)])

    return prop(x, src3, dst3)


def _tc_prepare(counts, features):
    def body(counts_ref, feat_ref, h_ref, no_ref, ni_ref):
        c = counts_ref[...]
        deg_o = c[0, 0, :, 0:1] + c[1, 0, :, 0:1]
        deg_i = c[0, 1, :, 0:1] + c[1, 1, :, 0:1]
        no = lax.rsqrt(jnp.maximum(deg_o, 1.0))
        ni = lax.rsqrt(jnp.maximum(deg_i, 1.0))
        h_ref[...] = feat_ref[...] * no
        no_ref[...] = no
        ni_ref[...] = ni

    return pl.pallas_call(
        body,
        out_shape=(
            jax.ShapeDtypeStruct((N, D), jnp.float32),
            jax.ShapeDtypeStruct((N, 1), jnp.float32),
            jax.ShapeDtypeStruct((N, 1), jnp.float32),
        ),
        grid=(N // _RB,),
        in_specs=[
            pl.BlockSpec((NC, 2, _RB, 16), lambda i: (0, 0, i, 0)),
            pl.BlockSpec((_RB, D), lambda i: (i, 0)),
        ],
        out_specs=(
            pl.BlockSpec((_RB, D), lambda i: (i, 0)),
            pl.BlockSpec((_RB, 1), lambda i: (i, 0)),
            pl.BlockSpec((_RB, 1), lambda i: (i, 0)),
        ),
    )(counts, features)


_RB = 2000


def _w_spec(shape):
    return pl.BlockSpec(shape, lambda i: (0,) * len(shape))


def _tc_mid(p, norm_in, norm_out, W10, b10, W11, b11, W20, W21):
    def body(p_ref, ni_ref, no_ref, W10_r, b10_r, W11_r, b11_r, W20_r, W21_r,
             q_ref):
        agg = (p_ref[0] + p_ref[1]) * ni_ref[...]
        o0 = jnp.maximum(
            jnp.dot(agg, W10_r[...], preferred_element_type=jnp.float32,
                    precision=_HIGH) + b10_r[...], 0.0)
        o1 = jnp.maximum(
            jnp.dot(agg, W11_r[...], preferred_element_type=jnp.float32,
                    precision=_HIGH) + b11_r[...], 0.0)
        q0 = jnp.dot(o0, W20_r[...], preferred_element_type=jnp.float32,
                     precision=_HIGH)
        q1 = jnp.dot(o1, W21_r[...], preferred_element_type=jnp.float32,
                     precision=_HIGH)
        q_ref[...] = jnp.concatenate([q0, q1], axis=1) * no_ref[...]

    return pl.pallas_call(
        body,
        grid=(N // _RB,),
        in_specs=[
            pl.BlockSpec((NC, _RB, D), lambda i: (0, i, 0)),
            pl.BlockSpec((_RB, 1), lambda i: (i, 0)),
            pl.BlockSpec((_RB, 1), lambda i: (i, 0)),
            _w_spec((D, D)), _w_spec((1, D)),
            _w_spec((D, D)), _w_spec((1, D)),
            _w_spec((D, D // 2)), _w_spec((D, D // 2)),
        ],
        out_specs=pl.BlockSpec((_RB, D), lambda i: (i, 0)),
        out_shape=jax.ShapeDtypeStruct((N, D), jnp.float32),
    )(p, norm_in, norm_out, W10, b10, W11, b11, W20, W21)


def _tc_final(p2, norm_in, features, b2cat, ln_g, ln_b, Wf1, bf1, Wf2, bf2):
    def body(p_ref, ni_ref, feat_ref, b2_r, g_r, b_r, Wf1_r, bf1_r, Wf2_r,
             bf2_r, out_ref):
        hcat = (p_ref[0] + p_ref[1]) * ni_ref[...] + b2_r[...]
        mu = jnp.mean(hcat, axis=1, keepdims=True)
        xc = hcat - mu
        var = jnp.mean(xc * xc, axis=1, keepdims=True)
        ln = xc * lax.rsqrt(var + 1e-5) * g_r[...] + b_r[...]
        h2 = feat_ref[...] + ln
        ff = jnp.maximum(
            jnp.dot(h2, Wf1_r[...], preferred_element_type=jnp.float32,
                    precision=_HIGH) + bf1_r[...], 0.0)
        ff = jnp.dot(ff, Wf2_r[...], preferred_element_type=jnp.float32,
                     precision=_HIGH) + bf2_r[...]
        out_ref[...] = h2 + ff

    return pl.pallas_call(
        body,
        grid=(N // _RB,),
        in_specs=[
            pl.BlockSpec((NC, _RB, D), lambda i: (0, i, 0)),
            pl.BlockSpec((_RB, 1), lambda i: (i, 0)),
            pl.BlockSpec((_RB, D), lambda i: (i, 0)),
            _w_spec((1, D)), _w_spec((1, D)), _w_spec((1, D)),
            _w_spec((D, D)), _w_spec((1, D)),
            _w_spec((D, D)), _w_spec((1, D)),
        ],
        out_specs=pl.BlockSpec((_RB, D), lambda i: (i, 0)),
        out_shape=jax.ShapeDtypeStruct((N, D), jnp.float32),
    )(p2, norm_in, features, b2cat, ln_g, ln_b, Wf1, bf1, Wf2, bf2)


def kernel(features, edge_index, W1_0, b1_0, W2_0, b2_0, W1_1, b1_1, W2_1,
           b2_1, ln_g, ln_b, Wf1, bf1, Wf2, bf2):
    src = edge_index[0]
    dst = edge_index[1]

    pad_i = jnp.arange(EPAD - E, dtype=jnp.int32)
    src3 = jnp.concatenate([src, pad_i % N]).reshape(NW, NCHUNK, K)
    dst3 = jnp.concatenate([dst, N + pad_i % PADR]).reshape(NW, NCHUNK, K)

    counts = _sc_degrees(src, dst)
    h_scaled, norm_out, norm_in = _tc_prepare(counts, features)

    p1 = _sc_propagate(h_scaled, src3, dst3)
    q = _tc_mid(p1, norm_in, norm_out,
                W1_0, b1_0.reshape(1, D), W1_1, b1_1.reshape(1, D), W2_0, W2_1)

    p2 = _sc_propagate(q, src3, dst3)
    b2cat = jnp.concatenate([b2_0, b2_1]).reshape(1, D)
    out = _tc_final(p2, norm_in, features, b2cat, ln_g.reshape(1, D),
                    ln_b.reshape(1, D), Wf1, bf1.reshape(1, D), Wf2,
                    bf2.reshape(1, D))
    return out


# speedup vs baseline: 3.2258x; 1.0563x over previous
import functools

import jax
import jax.numpy as jnp
from jax import lax
from jax.experimental import pallas as pl
from jax.experimental.pallas import tpu as pltpu
from jax.experimental.pallas import tpu_sc as plsc

N = 10000
D = 128
E = 320000
NC = 2
NS = 16
NW = NC * NS
EPW = E // NW
K = 128
NCHUNK = 80
EPWP = NCHUNK * K
EPAD = NW * EPWP
IH = 40
PADR = 512
NP = N + PADR
KH = 2000
NCHUNK_H = EPW // KH
ROWS_PT = 624

_HIGH = lax.Precision.HIGHEST


def _sc_mesh():
    return plsc.VectorSubcoreMesh(core_axis_name="c", subcore_axis_name="s")


def _sc_degrees(src, dst):
    @functools.partial(
        pl.kernel,
        out_type=jax.ShapeDtypeStruct((NC, 2, N, 16), jnp.float32),
        mesh=_sc_mesh(),
        compiler_params=pltpu.CompilerParams(use_tc_tiling_on_sc=False),
        scratch_types=[
            pltpu.VMEM((KH,), jnp.int32),
            pltpu.VMEM((KH,), jnp.int32),
            pltpu.VMEM((KH, 16), jnp.float32),
            pltpu.VMEM_SHARED((N, 16), jnp.float32),
            pltpu.VMEM_SHARED((N, 16), jnp.float32),
            pltpu.SemaphoreType.DMA,
            pltpu.SemaphoreType.DMA,
        ],
    )
    def hist(src_hbm, dst_hbm, out_hbm, idx_a, idx_b, ones_v, acc_s, acc_d,
             sem_a, sem_b):
        cid = lax.axis_index("c")
        sid = lax.axis_index("s")
        wid = sid * NC + cid
        one = jnp.ones((16,), jnp.float32)
        zero = jnp.zeros((16,), jnp.float32)

        @pl.loop(0, KH)
        def _(r):
            ones_v.at[r][...] = one

        row0 = sid * ROWS_PT

        @pl.loop(0, 64)
        def _(r):
            ones_v.at[r][...] = zero

        @pl.loop(0, 640, step=64)
        def _(r):
            pltpu.sync_copy(ones_v.at[pl.ds(0, 64)],
                            acc_s.at[pl.ds(row0 + r, 64)])
            pltpu.sync_copy(ones_v.at[pl.ds(0, 64)],
                            acc_d.at[pl.ds(row0 + r, 64)])

        @pl.loop(0, 64)
        def _(r):
            ones_v.at[r][...] = one

        base = wid * EPW
        pltpu.async_copy(src_hbm.at[pl.ds(base, KH)], idx_a, sem_a)
        pltpu.async_copy(dst_hbm.at[pl.ds(base, KH)], idx_b, sem_b)

        plsc.subcore_barrier()

        @pl.loop(0, NCHUNK_H)
        def _(chunk):
            off = base + (chunk + 1) * KH
            pltpu.make_async_copy(src_hbm.at[pl.ds(off - KH, KH)], idx_a,
                                  sem_a).wait()
            pltpu.sync_copy(ones_v, acc_s.at[idx_a], add=True)

            @pl.when(chunk < NCHUNK_H - 1)
            def _():
                pltpu.async_copy(src_hbm.at[pl.ds(off, KH)], idx_a, sem_a)

            pltpu.make_async_copy(dst_hbm.at[pl.ds(off - KH, KH)], idx_b,
                                  sem_b).wait()
            pltpu.sync_copy(ones_v, acc_d.at[idx_b], add=True)

            @pl.when(chunk < NCHUNK_H - 1)
            def _():
                pltpu.async_copy(dst_hbm.at[pl.ds(off, KH)], idx_b, sem_b)

        plsc.subcore_barrier()
        pltpu.sync_copy(acc_s.at[pl.ds(row0, ROWS_PT)],
                        out_hbm.at[cid, 0, pl.ds(row0, ROWS_PT)])
        pltpu.sync_copy(acc_d.at[pl.ds(row0, ROWS_PT)],
                        out_hbm.at[cid, 1, pl.ds(row0, ROWS_PT)])

        @pl.when(sid == NS - 1)
        def _():
            pltpu.sync_copy(acc_s.at[pl.ds(NS * ROWS_PT, N - NS * ROWS_PT)],
                            out_hbm.at[cid, 0, pl.ds(NS * ROWS_PT, N - NS * ROWS_PT)])
            pltpu.sync_copy(acc_d.at[pl.ds(NS * ROWS_PT, N - NS * ROWS_PT)],
                            out_hbm.at[cid, 1, pl.ds(NS * ROWS_PT, N - NS * ROWS_PT)])

    return hist(src, dst)


def _sc_propagate(x, src3, dst3):
    @functools.partial(
        pl.kernel,
        out_type=jax.ShapeDtypeStruct((NC, N, D), jnp.float32),
        mesh=_sc_mesh(),
        scratch_types=[
            pltpu.VMEM((IH, K), jnp.int32),
            pltpu.VMEM((IH, K), jnp.int32),
            pltpu.VMEM((K, D), jnp.float32),
            pltpu.VMEM((K, D), jnp.float32),
            pltpu.VMEM_SHARED((NP, D), jnp.float32),
            pltpu.SemaphoreType.DMA,
            pltpu.SemaphoreType.DMA,
        ],
    )
    def prop(x_hbm, src_hbm, dst_hbm, out_hbm, src_v, dst_v, rows0, rows1,
             acc_sh, sem0, sem1):
        cid = lax.axis_index("c")
        sid = lax.axis_index("s")
        wid = sid * NC + cid
        zero = jnp.zeros((16,), jnp.float32)

        @pl.loop(0, 64)
        def _(r):
            @pl.loop(0, D, step=16)
            def _(c0):
                rows0.at[r, pl.ds(c0, 16)][...] = zero

        row0 = sid * ROWS_PT

        @pl.loop(0, 640, step=64)
        def _(r):
            pltpu.sync_copy(rows0.at[pl.ds(0, 64)],
                            acc_sh.at[pl.ds(row0 + r, 64)])

        plsc.subcore_barrier()

        for phase in range(2):
            pltpu.sync_copy(src_hbm.at[wid, pl.ds(phase * IH, IH)], src_v)
            pltpu.sync_copy(dst_hbm.at[wid, pl.ds(phase * IH, IH)], dst_v)
            pltpu.async_copy(x_hbm.at[src_v.at[0]], rows0, sem0)

            @pl.loop(0, IH, step=2)
            def _(j):
                pltpu.async_copy(x_hbm.at[src_v.at[j + 1]], rows1, sem1)
                pltpu.make_async_copy(x_hbm.at[src_v.at[j]], rows0, sem0).wait()
                pltpu.sync_copy(rows0, acc_sh.at[dst_v.at[j]], add=True)

                @pl.when(j < IH - 2)
                def _():
                    pltpu.async_copy(x_hbm.at[src_v.at[j + 2]], rows0, sem0)

                pltpu.make_async_copy(x_hbm.at[src_v.at[j + 1]], rows1,
                                      sem1).wait()
                pltpu.sync_copy(rows1, acc_sh.at[dst_v.at[j + 1]], add=True)

        plsc.subcore_barrier()
        pltpu.sync_copy(acc_sh.at[pl.ds(row0, ROWS_PT)],
                        out_hbm.at[cid, pl.ds(row0, ROWS_PT)])

        @pl.when(sid == NS - 1)
        def _():
            pltpu.sync_copy(acc_sh.at[pl.ds(NS * ROWS_PT, N - NS * ROWS_PT)],
                            out_hbm.at[cid, pl.ds(NS * ROWS_PT, N - NS * ROWS_PT)])

    return prop(x, src3, dst3)


def _tc_prepare(counts, features):
    def body(counts_ref, feat_ref, h_ref, no_ref, ni_ref):
        c = counts_ref[...]
        deg_o = c[0, 0, :, 0:1] + c[1, 0, :, 0:1]
        deg_i = c[0, 1, :, 0:1] + c[1, 1, :, 0:1]
        no = lax.rsqrt(jnp.maximum(deg_o, 1.0))
        ni = lax.rsqrt(jnp.maximum(deg_i, 1.0))
        h_ref[...] = feat_ref[...] * no
        no_ref[...] = no
        ni_ref[...] = ni

    return pl.pallas_call(
        body,
        out_shape=(
            jax.ShapeDtypeStruct((N, D), jnp.float32),
            jax.ShapeDtypeStruct((N, 1), jnp.float32),
            jax.ShapeDtypeStruct((N, 1), jnp.float32),
        ),
        grid=(N // _RB,),
        in_specs=[
            pl.BlockSpec((NC, 2, _RB, 16), lambda i: (0, 0, i, 0)),
            pl.BlockSpec((_RB, D), lambda i: (i, 0)),
        ],
        out_specs=(
            pl.BlockSpec((_RB, D), lambda i: (i, 0)),
            pl.BlockSpec((_RB, 1), lambda i: (i, 0)),
            pl.BlockSpec((_RB, 1), lambda i: (i, 0)),
        ),
    )(counts, features)


_RB = 2000


def _w_spec(shape):
    return pl.BlockSpec(shape, lambda i: (0,) * len(shape))


def _tc_mid(p, norm_in, norm_out, W10, b10, W11, b11, W20, W21):
    def body(p_ref, ni_ref, no_ref, W10_r, b10_r, W11_r, b11_r, W20_r, W21_r,
             q_ref):
        agg = (p_ref[0] + p_ref[1]) * ni_ref[...]
        o0 = jnp.maximum(
            jnp.dot(agg, W10_r[...], preferred_element_type=jnp.float32,
                    precision=_HIGH) + b10_r[...], 0.0)
        o1 = jnp.maximum(
            jnp.dot(agg, W11_r[...], preferred_element_type=jnp.float32,
                    precision=_HIGH) + b11_r[...], 0.0)
        q0 = jnp.dot(o0, W20_r[...], preferred_element_type=jnp.float32,
                     precision=_HIGH)
        q1 = jnp.dot(o1, W21_r[...], preferred_element_type=jnp.float32,
                     precision=_HIGH)
        q_ref[...] = jnp.concatenate([q0, q1], axis=1) * no_ref[...]

    return pl.pallas_call(
        body,
        grid=(N // _RB,),
        in_specs=[
            pl.BlockSpec((NC, _RB, D), lambda i: (0, i, 0)),
            pl.BlockSpec((_RB, 1), lambda i: (i, 0)),
            pl.BlockSpec((_RB, 1), lambda i: (i, 0)),
            _w_spec((D, D)), _w_spec((1, D)),
            _w_spec((D, D)), _w_spec((1, D)),
            _w_spec((D, D // 2)), _w_spec((D, D // 2)),
        ],
        out_specs=pl.BlockSpec((_RB, D), lambda i: (i, 0)),
        out_shape=jax.ShapeDtypeStruct((N, D), jnp.float32),
    )(p, norm_in, norm_out, W10, b10, W11, b11, W20, W21)


def _tc_final(p2, norm_in, features, b2cat, ln_g, ln_b, Wf1, bf1, Wf2, bf2):
    def body(p_ref, ni_ref, feat_ref, b2_r, g_r, b_r, Wf1_r, bf1_r, Wf2_r,
             bf2_r, out_ref):
        hcat = (p_ref[0] + p_ref[1]) * ni_ref[...] + b2_r[...]
        mu = jnp.mean(hcat, axis=1, keepdims=True)
        xc = hcat - mu
        var = jnp.mean(xc * xc, axis=1, keepdims=True)
        ln = xc * lax.rsqrt(var + 1e-5) * g_r[...] + b_r[...]
        h2 = feat_ref[...] + ln
        ff = jnp.maximum(
            jnp.dot(h2, Wf1_r[...], preferred_element_type=jnp.float32,
                    precision=_HIGH) + bf1_r[...], 0.0)
        ff = jnp.dot(ff, Wf2_r[...], preferred_element_type=jnp.float32,
                     precision=_HIGH) + bf2_r[...]
        out_ref[...] = h2 + ff

    return pl.pallas_call(
        body,
        grid=(N // _RB,),
        in_specs=[
            pl.BlockSpec((NC, _RB, D), lambda i: (0, i, 0)),
            pl.BlockSpec((_RB, 1), lambda i: (i, 0)),
            pl.BlockSpec((_RB, D), lambda i: (i, 0)),
            _w_spec((1, D)), _w_spec((1, D)), _w_spec((1, D)),
            _w_spec((D, D)), _w_spec((1, D)),
            _w_spec((D, D)), _w_spec((1, D)),
        ],
        out_specs=pl.BlockSpec((_RB, D), lambda i: (i, 0)),
        out_shape=jax.ShapeDtypeStruct((N, D), jnp.float32),
    )(p2, norm_in, features, b2cat, ln_g, ln_b, Wf1, bf1, Wf2, bf2)


def kernel(features, edge_index, W1_0, b1_0, W2_0, b2_0, W1_1, b1_1, W2_1,
           b2_1, ln_g, ln_b, Wf1, bf1, Wf2, bf2):
    src = edge_index[0]
    dst = edge_index[1]

    pad_i = jnp.arange(EPAD - E, dtype=jnp.int32)
    src3 = jnp.concatenate([src, pad_i % N]).reshape(NW, NCHUNK, K)
    dst3 = jnp.concatenate([dst, N + pad_i % PADR]).reshape(NW, NCHUNK, K)

    counts = _sc_degrees(src, dst)
    h_scaled, norm_out, norm_in = _tc_prepare(counts, features)

    p1 = _sc_propagate(h_scaled, src3, dst3)
    q = _tc_mid(p1, norm_in, norm_out,
                W1_0, b1_0.reshape(1, D), W1_1, b1_1.reshape(1, D), W2_0, W2_1)

    p2 = _sc_propagate(q, src3, dst3)
    b2cat = jnp.concatenate([b2_0, b2_1]).reshape(1, D)
    out = _tc_final(p2, norm_in, features, b2cat, ln_g.reshape(1, D),
                    ln_b.reshape(1, D), Wf1, bf1.reshape(1, D), Wf2,
                    bf2.reshape(1, D))
    return out


# speedup vs baseline: 3.4317x; 1.0639x over previous
import functools

import jax
import jax.numpy as jnp
from jax import lax
from jax.experimental import pallas as pl
from jax.experimental.pallas import tpu as pltpu
from jax.experimental.pallas import tpu_sc as plsc

N = 10000
D = 128
E = 320000
NC = 2
NS = 16
NW = NC * NS
EPW = E // NW
K = 128
NCHUNK = 80
EPWP = NCHUNK * K
EPAD = NW * EPWP
IH = 40
PADR = 512
NP = N + PADR
KH = 2000
NCHUNK_H = EPW // KH
ROWS_PT = 624

_HIGH = lax.Precision.HIGHEST


def _sc_mesh():
    return plsc.VectorSubcoreMesh(core_axis_name="c", subcore_axis_name="s")


def _sc_degrees(src, dst):
    @functools.partial(
        pl.kernel,
        out_type=jax.ShapeDtypeStruct((NC, 2, N, 16), jnp.float32),
        mesh=_sc_mesh(),
        compiler_params=pltpu.CompilerParams(use_tc_tiling_on_sc=False),
        scratch_types=[
            pltpu.VMEM((KH,), jnp.int32),
            pltpu.VMEM((KH,), jnp.int32),
            pltpu.VMEM((KH, 16), jnp.float32),
            pltpu.VMEM_SHARED((N, 16), jnp.float32),
            pltpu.VMEM_SHARED((N, 16), jnp.float32),
            pltpu.SemaphoreType.DMA,
            pltpu.SemaphoreType.DMA,
        ],
    )
    def hist(src_hbm, dst_hbm, out_hbm, idx_a, idx_b, ones_v, acc_s, acc_d,
             sem_a, sem_b):
        cid = lax.axis_index("c")
        sid = lax.axis_index("s")
        wid = sid * NC + cid
        one = jnp.ones((16,), jnp.float32)
        zero = jnp.zeros((16,), jnp.float32)

        @pl.loop(0, KH)
        def _(r):
            ones_v.at[r][...] = one

        row0 = sid * ROWS_PT

        @pl.loop(0, 64)
        def _(r):
            ones_v.at[r][...] = zero

        @pl.loop(0, 640, step=64)
        def _(r):
            pltpu.sync_copy(ones_v.at[pl.ds(0, 64)],
                            acc_s.at[pl.ds(row0 + r, 64)])
            pltpu.sync_copy(ones_v.at[pl.ds(0, 64)],
                            acc_d.at[pl.ds(row0 + r, 64)])

        @pl.loop(0, 64)
        def _(r):
            ones_v.at[r][...] = one

        base = wid * EPW
        pltpu.async_copy(src_hbm.at[pl.ds(base, KH)], idx_a, sem_a)
        pltpu.async_copy(dst_hbm.at[pl.ds(base, KH)], idx_b, sem_b)

        plsc.subcore_barrier()

        @pl.loop(0, NCHUNK_H)
        def _(chunk):
            off = base + (chunk + 1) * KH
            pltpu.make_async_copy(src_hbm.at[pl.ds(off - KH, KH)], idx_a,
                                  sem_a).wait()
            pltpu.sync_copy(ones_v, acc_s.at[idx_a], add=True)

            @pl.when(chunk < NCHUNK_H - 1)
            def _():
                pltpu.async_copy(src_hbm.at[pl.ds(off, KH)], idx_a, sem_a)

            pltpu.make_async_copy(dst_hbm.at[pl.ds(off - KH, KH)], idx_b,
                                  sem_b).wait()
            pltpu.sync_copy(ones_v, acc_d.at[idx_b], add=True)

            @pl.when(chunk < NCHUNK_H - 1)
            def _():
                pltpu.async_copy(dst_hbm.at[pl.ds(off, KH)], idx_b, sem_b)

        plsc.subcore_barrier()
        pltpu.sync_copy(acc_s.at[pl.ds(row0, ROWS_PT)],
                        out_hbm.at[cid, 0, pl.ds(row0, ROWS_PT)])
        pltpu.sync_copy(acc_d.at[pl.ds(row0, ROWS_PT)],
                        out_hbm.at[cid, 1, pl.ds(row0, ROWS_PT)])

        @pl.when(sid == NS - 1)
        def _():
            pltpu.sync_copy(acc_s.at[pl.ds(NS * ROWS_PT, N - NS * ROWS_PT)],
                            out_hbm.at[cid, 0, pl.ds(NS * ROWS_PT, N - NS * ROWS_PT)])
            pltpu.sync_copy(acc_d.at[pl.ds(NS * ROWS_PT, N - NS * ROWS_PT)],
                            out_hbm.at[cid, 1, pl.ds(NS * ROWS_PT, N - NS * ROWS_PT)])

    return hist(src, dst)


def _sc_propagate(x, src3, dst3):
    @functools.partial(
        pl.kernel,
        out_type=jax.ShapeDtypeStruct((NC, N, D), jnp.float32),
        mesh=_sc_mesh(),
        scratch_types=[
            pltpu.VMEM((IH, K), jnp.int32),
            pltpu.VMEM((IH, K), jnp.int32),
            pltpu.VMEM((K, D), jnp.float32),
            pltpu.VMEM((K, D), jnp.float32),
            pltpu.VMEM_SHARED((NP, D), jnp.float32),
            pltpu.SemaphoreType.DMA,
            pltpu.SemaphoreType.DMA,
        ],
    )
    def prop(x_hbm, src_hbm, dst_hbm, out_hbm, src_v, dst_v, rows0, rows1,
             acc_sh, sem0, sem1):
        cid = lax.axis_index("c")
        sid = lax.axis_index("s")
        wid = sid * NC + cid
        zero = jnp.zeros((16,), jnp.float32)

        @pl.loop(0, 64)
        def _(r):
            @pl.loop(0, D, step=16)
            def _(c0):
                rows0.at[r, pl.ds(c0, 16)][...] = zero

        row0 = sid * ROWS_PT

        @pl.loop(0, 640, step=64)
        def _(r):
            pltpu.sync_copy(rows0.at[pl.ds(0, 64)],
                            acc_sh.at[pl.ds(row0 + r, 64)])

        plsc.subcore_barrier()

        for phase in range(2):
            pltpu.sync_copy(src_hbm.at[wid, pl.ds(phase * IH, IH)], src_v)
            pltpu.sync_copy(dst_hbm.at[wid, pl.ds(phase * IH, IH)], dst_v)
            pltpu.async_copy(x_hbm.at[src_v.at[0]], rows0, sem0)

            @pl.loop(0, IH, step=2)
            def _(j):
                pltpu.async_copy(x_hbm.at[src_v.at[j + 1]], rows1, sem1)
                pltpu.make_async_copy(x_hbm.at[src_v.at[j]], rows0, sem0).wait()
                pltpu.sync_copy(rows0, acc_sh.at[dst_v.at[j]], add=True)

                @pl.when(j < IH - 2)
                def _():
                    pltpu.async_copy(x_hbm.at[src_v.at[j + 2]], rows0, sem0)

                pltpu.make_async_copy(x_hbm.at[src_v.at[j + 1]], rows1,
                                      sem1).wait()
                pltpu.sync_copy(rows1, acc_sh.at[dst_v.at[j + 1]], add=True)

        plsc.subcore_barrier()
        pltpu.sync_copy(acc_sh.at[pl.ds(row0, ROWS_PT)],
                        out_hbm.at[cid, pl.ds(row0, ROWS_PT)])

        @pl.when(sid == NS - 1)
        def _():
            pltpu.sync_copy(acc_sh.at[pl.ds(NS * ROWS_PT, N - NS * ROWS_PT)],
                            out_hbm.at[cid, pl.ds(NS * ROWS_PT, N - NS * ROWS_PT)])

    return prop(x, src3, dst3)


def _tc_prepare(counts, features):
    def body(counts_ref, feat_ref, h_ref, no_ref, ni_ref):
        c = counts_ref[...]
        deg_o = c[0, 0, :, 0:1] + c[1, 0, :, 0:1]
        deg_i = c[0, 1, :, 0:1] + c[1, 1, :, 0:1]
        no = lax.rsqrt(jnp.maximum(deg_o, 1.0))
        ni = lax.rsqrt(jnp.maximum(deg_i, 1.0))
        h_ref[...] = feat_ref[...] * no
        no_ref[...] = no
        ni_ref[...] = ni

    return pl.pallas_call(
        body,
        out_shape=(
            jax.ShapeDtypeStruct((N, D), jnp.float32),
            jax.ShapeDtypeStruct((N, 1), jnp.float32),
            jax.ShapeDtypeStruct((N, 1), jnp.float32),
        ),
        grid=(N // _RB,),
        in_specs=[
            pl.BlockSpec((NC, 2, _RB, 16), lambda i: (0, 0, i, 0)),
            pl.BlockSpec((_RB, D), lambda i: (i, 0)),
        ],
        out_specs=(
            pl.BlockSpec((_RB, D), lambda i: (i, 0)),
            pl.BlockSpec((_RB, 1), lambda i: (i, 0)),
            pl.BlockSpec((_RB, 1), lambda i: (i, 0)),
        ),
    )(counts, features)


_RB = 2000


def _w_spec(shape):
    return pl.BlockSpec(shape, lambda i: (0,) * len(shape))


def _tc_mid(p, norm_in, norm_out, W10, b10, W11, b11, W20, W21):
    def body(p_ref, ni_ref, no_ref, W10_r, b10_r, W11_r, b11_r, W20_r, W21_r,
             q_ref):
        agg = (p_ref[0] + p_ref[1]) * ni_ref[...]
        o0 = jnp.maximum(
            jnp.dot(agg, W10_r[...], preferred_element_type=jnp.float32) + b10_r[...], 0.0)
        o1 = jnp.maximum(
            jnp.dot(agg, W11_r[...], preferred_element_type=jnp.float32) + b11_r[...], 0.0)
        q0 = jnp.dot(o0, W20_r[...], preferred_element_type=jnp.float32)
        q1 = jnp.dot(o1, W21_r[...], preferred_element_type=jnp.float32)
        q_ref[...] = jnp.concatenate([q0, q1], axis=1) * no_ref[...]

    return pl.pallas_call(
        body,
        grid=(N // _RB,),
        in_specs=[
            pl.BlockSpec((NC, _RB, D), lambda i: (0, i, 0)),
            pl.BlockSpec((_RB, 1), lambda i: (i, 0)),
            pl.BlockSpec((_RB, 1), lambda i: (i, 0)),
            _w_spec((D, D)), _w_spec((1, D)),
            _w_spec((D, D)), _w_spec((1, D)),
            _w_spec((D, D // 2)), _w_spec((D, D // 2)),
        ],
        out_specs=pl.BlockSpec((_RB, D), lambda i: (i, 0)),
        out_shape=jax.ShapeDtypeStruct((N, D), jnp.float32),
    )(p, norm_in, norm_out, W10, b10, W11, b11, W20, W21)


def _tc_final(p2, norm_in, features, b2cat, ln_g, ln_b, Wf1, bf1, Wf2, bf2):
    def body(p_ref, ni_ref, feat_ref, b2_r, g_r, b_r, Wf1_r, bf1_r, Wf2_r,
             bf2_r, out_ref):
        hcat = (p_ref[0] + p_ref[1]) * ni_ref[...] + b2_r[...]
        mu = jnp.mean(hcat, axis=1, keepdims=True)
        xc = hcat - mu
        var = jnp.mean(xc * xc, axis=1, keepdims=True)
        ln = xc * lax.rsqrt(var + 1e-5) * g_r[...] + b_r[...]
        h2 = feat_ref[...] + ln
        ff = jnp.maximum(
            jnp.dot(h2, Wf1_r[...], preferred_element_type=jnp.float32) + bf1_r[...], 0.0)
        ff = jnp.dot(ff, Wf2_r[...], preferred_element_type=jnp.float32) + bf2_r[...]
        out_ref[...] = h2 + ff

    return pl.pallas_call(
        body,
        grid=(N // _RB,),
        in_specs=[
            pl.BlockSpec((NC, _RB, D), lambda i: (0, i, 0)),
            pl.BlockSpec((_RB, 1), lambda i: (i, 0)),
            pl.BlockSpec((_RB, D), lambda i: (i, 0)),
            _w_spec((1, D)), _w_spec((1, D)), _w_spec((1, D)),
            _w_spec((D, D)), _w_spec((1, D)),
            _w_spec((D, D)), _w_spec((1, D)),
        ],
        out_specs=pl.BlockSpec((_RB, D), lambda i: (i, 0)),
        out_shape=jax.ShapeDtypeStruct((N, D), jnp.float32),
    )(p2, norm_in, features, b2cat, ln_g, ln_b, Wf1, bf1, Wf2, bf2)


def kernel(features, edge_index, W1_0, b1_0, W2_0, b2_0, W1_1, b1_1, W2_1,
           b2_1, ln_g, ln_b, Wf1, bf1, Wf2, bf2):
    src = edge_index[0]
    dst = edge_index[1]

    pad_i = jnp.arange(EPAD - E, dtype=jnp.int32)
    src3 = jnp.concatenate([src, pad_i % N]).reshape(NW, NCHUNK, K)
    dst3 = jnp.concatenate([dst, N + pad_i % PADR]).reshape(NW, NCHUNK, K)

    counts = _sc_degrees(src, dst)
    h_scaled, norm_out, norm_in = _tc_prepare(counts, features)

    p1 = _sc_propagate(h_scaled, src3, dst3)
    q = _tc_mid(p1, norm_in, norm_out,
                W1_0, b1_0.reshape(1, D), W1_1, b1_1.reshape(1, D), W2_0, W2_1)

    p2 = _sc_propagate(q, src3, dst3)
    b2cat = jnp.concatenate([b2_0, b2_1]).reshape(1, D)
    out = _tc_final(p2, norm_in, features, b2cat, ln_g.reshape(1, D),
                    ln_b.reshape(1, D), Wf1, bf1.reshape(1, D), Wf2,
                    bf2.reshape(1, D))
    return out
